# Initial kernel scaffold; baseline (speedup 1.0000x reference)
import jax
import jax.numpy as jnp
from jax.experimental import pallas as pl


def kernel(x, edge_index, W0, b0, W1, b1, W2, b2, gamma0, beta0, gamma1, beta1):
    raise NotImplementedError("write your pallas kernel here")


# speedup vs baseline: 4.8547x; 4.8547x over previous
import functools

import jax
import jax.numpy as jnp
from jax import lax
from jax.experimental import pallas as pl
from jax.experimental.pallas import tpu as pltpu
from jax.experimental.pallas import tpu_sc as plsc

N_NODES = 10000
N_EDGES = 320000
D = 128
EPS_BN = 1e-5

NC = 2
NS = 16
NW = NC * NS
EDGES_PER_WORKER = N_EDGES // NW
CHUNK = 80
NCHUNKS = EDGES_PER_WORKER // CHUNK
N_PAD = 10240
ROWS_PER_TILE = N_PAD // NS
ZROWS = 16


def _sc_agg_body(with_deg, *refs):
    if with_deg:
        (h_hbm, src_hbm, dst_hbm, out_hbm, deg_hbm,
         sidx, didx, rows, zeros_v, acc_sh, sem) = refs
    else:
        (h_hbm, src_hbm, dst_hbm, out_hbm,
         sidx, didx, rows, zeros_v, acc_sh, sem) = refs

    c = lax.axis_index("c")
    s = lax.axis_index("s")
    w = c * NS + s
    wbase = w * EDGES_PER_WORKER
    row0 = s * ROWS_PER_TILE

    zero16 = jnp.zeros((16,), jnp.float32)
    for r in range(ZROWS):
        for q in range(D // 16):
            zeros_v[r, pl.ds(q * 16, 16)] = zero16

    def zero_acc():
        for i in range(ROWS_PER_TILE // ZROWS):
            pltpu.sync_copy(zeros_v, acc_sh.at[pl.ds(row0 + i * ZROWS, ZROWS)])

    if with_deg:
        one16 = jnp.ones((16,), jnp.float32)
        for r in range(CHUNK):
            for q in range(D // 16):
                rows[r, pl.ds(q * 16, 16)] = one16
        zero_acc()
        plsc.subcore_barrier()

        @pl.loop(0, NCHUNKS)
        def _(j):
            base = pl.multiple_of(wbase + j * CHUNK, CHUNK)
            pltpu.sync_copy(dst_hbm.at[pl.ds(base, CHUNK)], didx)
            pltpu.sync_copy(rows, acc_sh.at[didx], add=True)

        plsc.subcore_barrier()
        pltpu.sync_copy(acc_sh.at[pl.ds(row0, ROWS_PER_TILE)],
                        deg_hbm.at[c, pl.ds(row0, ROWS_PER_TILE)])

    zero_acc()
    plsc.subcore_barrier()

    @pl.loop(0, NCHUNKS)
    def _(j):
        base = pl.multiple_of(wbase + j * CHUNK, CHUNK)
        pltpu.sync_copy(src_hbm.at[pl.ds(base, CHUNK)], sidx)
        pltpu.sync_copy(dst_hbm.at[pl.ds(base, CHUNK)], didx)
        pltpu.async_copy(h_hbm.at[sidx], rows, sem).wait()
        pltpu.sync_copy(rows, acc_sh.at[didx], add=True)

    plsc.subcore_barrier()
    pltpu.sync_copy(acc_sh.at[pl.ds(row0, ROWS_PER_TILE)],
                    out_hbm.at[c, pl.ds(row0, ROWS_PER_TILE)])


def _make_sc_agg(with_deg):
    mesh = plsc.VectorSubcoreMesh(core_axis_name="c", subcore_axis_name="s",
                                  num_cores=NC, num_subcores=NS)
    out_type = [jax.ShapeDtypeStruct((NC, N_PAD, D), jnp.float32)]
    if with_deg:
        out_type.append(jax.ShapeDtypeStruct((NC, N_PAD, D), jnp.float32))
    scratch = [
        pltpu.VMEM((CHUNK,), jnp.int32),
        pltpu.VMEM((CHUNK,), jnp.int32),
        pltpu.VMEM((CHUNK, D), jnp.float32),
        pltpu.VMEM((ZROWS, D), jnp.float32),
        pltpu.VMEM_SHARED((N_PAD, D), jnp.float32),
        pltpu.SemaphoreType.DMA,
    ]
    return pl.kernel(
        functools.partial(_sc_agg_body, with_deg),
        out_type=out_type,
        mesh=mesh,
        scratch_types=scratch,
    )


def _tc_layer_body(first, bn, p_ref, h_ref, aux_ref, w_ref, b_ref,
                   gamma_ref, beta_ref, o_ref, inv_ref=None):
    if first:
        deg = aux_ref[0, :N_NODES, 0:1] + aux_ref[1, :N_NODES, 0:1]
        inv = 1.0 / jnp.maximum(deg, 1.0)
        inv_ref[...] = inv
    else:
        inv = aux_ref[...]
    p = p_ref[0, :N_NODES, :] + p_ref[1, :N_NODES, :]
    t = h_ref[...] + p * inv
    y = lax.dot_general(t, w_ref[...], (((1,), (1,)), ((), ())),
                        preferred_element_type=jnp.float32) + b_ref[...]
    if bn:
        mu = jnp.mean(y, axis=0, keepdims=True)
        var = jnp.mean((y - mu) * (y - mu), axis=0, keepdims=True)
        y = gamma_ref[...] * (y - mu) * lax.rsqrt(var + EPS_BN) + beta_ref[...]
        y = jnp.maximum(y, 0.0)
    o_ref[...] = y


def _make_tc_layer(first, bn):
    out_shape = [jax.ShapeDtypeStruct((N_NODES, D), jnp.float32)]
    if first:
        out_shape.append(jax.ShapeDtypeStruct((N_NODES, 1), jnp.float32))
    return pl.pallas_call(
        functools.partial(_tc_layer_body, first, bn),
        out_shape=out_shape,
    )


def kernel(x, edge_index, W0, b0, W1, b1, W2, b2,
           gamma0, beta0, gamma1, beta1):
    src = edge_index[0].astype(jnp.int32)
    dst = edge_index[1].astype(jnp.int32)

    sc_agg_deg = _make_sc_agg(True)
    sc_agg = _make_sc_agg(False)
    tc_first = _make_tc_layer(True, True)
    tc_mid = _make_tc_layer(False, True)
    tc_last = _make_tc_layer(False, False)

    b0r = b0.reshape(1, D)
    b1r = b1.reshape(1, D)
    b2r = b2.reshape(1, D)
    g0 = gamma0.reshape(1, D)
    g1 = gamma1.reshape(1, D)
    be0 = beta0.reshape(1, D)
    be1 = beta1.reshape(1, D)

    p1, degp = sc_agg_deg(x, src, dst)
    h1, inv = tc_first(p1, x, degp, W0, b0r, g0, be0)
    (p2,) = sc_agg(h1, src, dst)
    (h2,) = tc_mid(p2, h1, inv, W1, b1r, g1, be1)
    (p3,) = sc_agg(h2, src, dst)
    (out,) = tc_last(p3, h2, inv, W2, b2r, g1, be1)
    return out
